# baseline (speedup 1.0000x reference)
import jax
import jax.numpy as jnp
from jax.experimental import pallas as pl

_B = 16384
_D = 256
_BLOCK = 8192


def _body(pe_ref, out_ref):
    out_ref[...] = jnp.broadcast_to(pe_ref[...][None], out_ref.shape)


def kernel(batch_size, pos_embed):
    out = pl.pallas_call(
        _body,
        grid=(_B // _BLOCK,),
        in_specs=[pl.BlockSpec((1, _D), lambda i: (0, 0))],
        out_specs=pl.BlockSpec((_BLOCK, 1, _D), lambda i: (i, 0, 0)),
        out_shape=jax.ShapeDtypeStruct((_B, 1, _D), jnp.float32),
    )(pos_embed)
    return out


# speedup vs baseline: 1.2918x; 1.2918x over previous
import jax
import jax.numpy as jnp
from jax.experimental import pallas as pl
from jax.experimental.pallas import tpu as pltpu

_B = 16384
_D = 256
_CHUNK = 4096
_T = _B // _CHUNK


def _body(pe_ref, out_hbm, buf, sem):
    buf[...] = jnp.broadcast_to(pe_ref[...], buf.shape)
    copies = [
        pltpu.make_async_copy(buf, out_hbm.at[pl.ds(t * _CHUNK, _CHUNK), 0, :], sem)
        for t in range(_T)
    ]
    for c in copies:
        c.start()
    for c in copies:
        c.wait()


def kernel(batch_size, pos_embed):
    return pl.pallas_call(
        _body,
        in_specs=[pl.BlockSpec(memory_space=pltpu.VMEM)],
        out_specs=pl.BlockSpec(memory_space=pltpu.HBM),
        out_shape=jax.ShapeDtypeStruct((_B, 1, _D), jnp.float32),
        scratch_shapes=[
            pltpu.VMEM((_CHUNK, _D), jnp.float32),
            pltpu.SemaphoreType.DMA,
        ],
    )(pos_embed)


# speedup vs baseline: 1.3183x; 1.0205x over previous
import jax
import jax.numpy as jnp
from jax.experimental import pallas as pl
from jax.experimental.pallas import tpu as pltpu

_B = 16384
_D = 256
_CHUNK = 2048
_T = _B // _CHUNK


def _body(pe_ref, out_hbm, buf, sem):
    buf[...] = jnp.broadcast_to(pe_ref[...], buf.shape)
    copies = [
        pltpu.make_async_copy(buf, out_hbm.at[pl.ds(t * _CHUNK, _CHUNK), 0, :], sem)
        for t in range(_T)
    ]
    for c in copies:
        c.start()
    for c in copies:
        c.wait()


def kernel(batch_size, pos_embed):
    return pl.pallas_call(
        _body,
        in_specs=[pl.BlockSpec(memory_space=pltpu.VMEM)],
        out_specs=pl.BlockSpec(memory_space=pltpu.HBM),
        out_shape=jax.ShapeDtypeStruct((_B, 1, _D), jnp.float32),
        scratch_shapes=[
            pltpu.VMEM((_CHUNK, _D), jnp.float32),
            pltpu.SemaphoreType.DMA,
        ],
    )(pos_embed)


# speedup vs baseline: 1.3368x; 1.0140x over previous
import jax
import jax.numpy as jnp
from jax.experimental import pallas as pl
from jax.experimental.pallas import tpu as pltpu

_B = 16384
_D = 256
_CHUNK = 1024
_T = _B // _CHUNK


def _body(pe_ref, out_hbm, buf, sem):
    buf[...] = jnp.broadcast_to(pe_ref[...], buf.shape)
    copies = [
        pltpu.make_async_copy(buf, out_hbm.at[pl.ds(t * _CHUNK, _CHUNK), 0, :], sem)
        for t in range(_T)
    ]
    for c in copies:
        c.start()
    for c in copies:
        c.wait()


def kernel(batch_size, pos_embed):
    return pl.pallas_call(
        _body,
        in_specs=[pl.BlockSpec(memory_space=pltpu.VMEM)],
        out_specs=pl.BlockSpec(memory_space=pltpu.HBM),
        out_shape=jax.ShapeDtypeStruct((_B, 1, _D), jnp.float32),
        scratch_shapes=[
            pltpu.VMEM((_CHUNK, _D), jnp.float32),
            pltpu.SemaphoreType.DMA,
        ],
    )(pos_embed)


# speedup vs baseline: 1.3409x; 1.0031x over previous
import jax
import jax.numpy as jnp
from jax.experimental import pallas as pl
from jax.experimental.pallas import tpu as pltpu

_B = 16384
_D = 256
_CHUNK = 512
_T = _B // _CHUNK


def _body(pe_ref, out_hbm, buf, sem):
    buf[...] = jnp.broadcast_to(pe_ref[...], buf.shape)
    copies = [
        pltpu.make_async_copy(buf, out_hbm.at[pl.ds(t * _CHUNK, _CHUNK), 0, :], sem)
        for t in range(_T)
    ]
    for c in copies:
        c.start()
    for c in copies:
        c.wait()


def kernel(batch_size, pos_embed):
    return pl.pallas_call(
        _body,
        in_specs=[pl.BlockSpec(memory_space=pltpu.VMEM)],
        out_specs=pl.BlockSpec(memory_space=pltpu.HBM),
        out_shape=jax.ShapeDtypeStruct((_B, 1, _D), jnp.float32),
        scratch_shapes=[
            pltpu.VMEM((_CHUNK, _D), jnp.float32),
            pltpu.SemaphoreType.DMA,
        ],
    )(pos_embed)
